# baseline (speedup 1.0000x reference)
import jax
import jax.numpy as jnp
from jax import lax
from jax.experimental import pallas as pl
from jax.experimental.pallas import tpu as pltpu
from jax.experimental.pallas import tpu_sc as plsc
import functools

_NUM_ITEMS = 100000
_HIST = 20
_BATCH = 1024
_N = _BATCH * _NUM_ITEMS
_NC = 2
_NS = 16
_L = 16
_NW = _NC * _NS
_ROWS_PER_W = _BATCH // _NW
_HP = 32
_ZGRID = 25


def _tc_zero_body(o_ref):
    o_ref[...] = jnp.zeros_like(o_ref)


def _sc_body(uid_hbm, upi_hbm, p_hbm, buf, uid_v, items_v, idx_v, val_v,
             gsem, dsem):
    c = lax.axis_index("c")
    s = lax.axis_index("s")
    wid = s * _NC + c
    base = wid * _ROWS_PER_W

    pltpu.sync_copy(uid_hbm.at[pl.ds(base, _ROWS_PER_W)], uid_v)
    pltpu.async_copy(upi_hbm.at[uid_v], items_v, gsem).wait()
    pltpu.sync_copy(p_hbm, val_v)
    for t in range(8):
        val_v[pl.ds(t * _L, _L)] = val_v[pl.ds(t * _L, _L)] + 100.0

    for r in range(_ROWS_PER_W):
        fo = (base + r) * _NUM_ITEMS
        j, k = divmod(r, 4)
        idx_v[j, pl.ds(k * 32, _L)] = items_v[r, pl.ds(0, _L)] + fo
        idx_v[j, pl.ds(k * 32 + _L, _L)] = items_v[r, pl.ds(4, _L)] + fo

    for j in range(8):
        pltpu.async_copy(val_v, buf.at[idx_v.at[j]], dsem)
    for j in range(8):
        pltpu.make_async_copy(val_v, buf.at[idx_v.at[j]], dsem).wait()


@jax.jit
def kernel(users_ids, users_pos_items, param):
    p128 = jnp.broadcast_to(param.astype(jnp.float32), (8 * _L,))
    upi_p = jnp.pad(users_pos_items.astype(jnp.int32),
                    ((0, 0), (0, _HP - _HIST)))

    zeros = pl.pallas_call(
        _tc_zero_body,
        out_shape=jax.ShapeDtypeStruct((_N,), jnp.float32),
        grid=(_ZGRID,),
        out_specs=pl.BlockSpec((_N // _ZGRID,), lambda i: (i,)),
    )()
    buf = jax.new_ref(zeros)

    mesh = plsc.VectorSubcoreMesh(
        core_axis_name="c", subcore_axis_name="s", num_cores=_NC,
        num_subcores=_NS)
    scatter = functools.partial(
        pl.kernel,
        out_type=(),
        mesh=mesh,
        compiler_params=pltpu.CompilerParams(
            needs_layout_passes=False, use_tc_tiling_on_sc=False),
        scratch_types=[
            pltpu.VMEM((_ROWS_PER_W,), jnp.int32),
            pltpu.VMEM((_ROWS_PER_W, _HP), jnp.int32),
            pltpu.VMEM((8, 128), jnp.int32),
            pltpu.VMEM((128,), jnp.float32),
            pltpu.SemaphoreType.DMA,
            pltpu.SemaphoreType.DMA,
        ],
    )(_sc_body)
    scatter(users_ids.astype(jnp.int32), upi_p, p128, buf)
    return jax.freeze(buf).reshape(_BATCH, _NUM_ITEMS)


# speedup vs baseline: 2.0121x; 2.0121x over previous
import jax
import jax.numpy as jnp
from jax import lax
from jax.experimental import pallas as pl
from jax.experimental.pallas import tpu as pltpu
from jax.experimental.pallas import tpu_sc as plsc
import functools

_NUM_ITEMS = 100000
_HIST = 20
_BATCH = 1024
_NC = 2
_NS = 16
_L = 16
_NW = _NC * _NS
_ROWS_PER_W = _BATCH // _NW
_HP = 128


def _sc_body(uid_hbm, upi_hbm, p_hbm, out_hbm, uid_v, items_v, p_v, zbuf,
             gsem):
    c = lax.axis_index("c")
    s = lax.axis_index("s")
    wid = s * _NC + c
    base = wid * _ROWS_PER_W

    pltpu.sync_copy(uid_hbm.at[pl.ds(base, _ROWS_PER_W)], uid_v)
    pltpu.async_copy(upi_hbm.at[uid_v], items_v, gsem).wait()
    pltpu.sync_copy(p_hbm, p_v)

    vval = p_v[...] + 100.0
    vzero = jnp.zeros((_L,), jnp.float32)

    def zfill(j, carry):
        for k in range(10):
            zbuf[pl.ds((j * 10 + k) * _L, _L)] = vzero
        return carry

    lax.fori_loop(0, 625, zfill, 0)

    mask_hi = lax.iota(jnp.int32, _L) >= 12

    def row(i, carry):
        idx0 = items_v[i, pl.ds(0, _L)]
        idx1 = items_v[i, pl.ds(4, _L)]
        plsc.store_scatter(zbuf, [idx0], vval)
        plsc.store_scatter(zbuf, [idx1], vval, mask=mask_hi)
        pltpu.sync_copy(zbuf, out_hbm.at[base + i])
        plsc.store_scatter(zbuf, [idx0], vzero)
        plsc.store_scatter(zbuf, [idx1], vzero, mask=mask_hi)
        return carry

    lax.fori_loop(0, _ROWS_PER_W, row, 0)


@jax.jit
def kernel(users_ids, users_pos_items, param):
    mesh = plsc.VectorSubcoreMesh(
        core_axis_name="c", subcore_axis_name="s", num_cores=_NC,
        num_subcores=_NS)
    p16 = jnp.broadcast_to(param.astype(jnp.float32), (_L,))
    upi_p = jnp.pad(users_pos_items.astype(jnp.int32),
                    ((0, 0), (0, _HP - _HIST)))
    run = functools.partial(
        pl.kernel,
        out_type=jax.ShapeDtypeStruct((_BATCH, _NUM_ITEMS), jnp.float32),
        mesh=mesh,
        compiler_params=pltpu.CompilerParams(
            needs_layout_passes=False, use_tc_tiling_on_sc=True),
        scratch_types=[
            pltpu.VMEM((_ROWS_PER_W,), jnp.int32),
            pltpu.VMEM((_ROWS_PER_W, _HP), jnp.int32),
            pltpu.VMEM((_L,), jnp.float32),
            pltpu.VMEM((_NUM_ITEMS,), jnp.float32),
            pltpu.SemaphoreType.DMA,
        ],
    )(_sc_body)
    return run(users_ids.astype(jnp.int32), upi_p, p16)
